# baseline (speedup 1.0000x reference)
import functools

import jax
import jax.numpy as jnp
from jax import lax
from jax.experimental import pallas as pl
from jax.experimental.pallas import tpu as pltpu
from jax.experimental.pallas import tpu_sc as plsc

B = 8
V = 10000
S = 9
F = 128
OUT = 128

B2 = B // 2
NC = 2
NS = 16
NW = NC * NS
CH = 128
VH = V // 2
KPH = 5
VPH = KPH * 8 * CH


def _sc_gather(xpk, idxp):
    S_, KPW_, _, CH_ = idxp.shape
    VP_ = KPW_ * 8 * CH_
    JOBW = S_ * KPW_
    NB = 5 if JOBW % 6 else 6
    mesh = plsc.VectorSubcoreMesh(core_axis_name="c", subcore_axis_name="s")

    @functools.partial(
        pl.kernel,
        mesh=mesh,
        out_type=jax.ShapeDtypeStruct((S_, B2, VP_, F), jnp.int32),
        scratch_types=[
            pltpu.VMEM((S_, KPW_, CH_), jnp.int32),
            [pltpu.VMEM((CH_, F), jnp.int32) for _ in range(NB)],
            [pltpu.SemaphoreType.DMA for _ in range(NB)],
            [pltpu.SemaphoreType.DMA for _ in range(NB)],
        ],
    )
    def k(xpk_hbm, idxp_hbm, g_hbm, idxall, rows, gsem, wsem):
        cid = lax.axis_index("c")
        sid = lax.axis_index("s")
        wid = sid * NC + cid
        pp = wid % B2
        base_q = wid // B2
        off = pp * (V + 1)

        pltpu.sync_copy(idxp_hbm.at[:, :, base_q, :], idxall)

        def add_off(a, carry):
            s = a // KPW_
            kk = a % KPW_
            for i in range(CH_ // 16):
                sl = pl.ds(i * 16, 16)
                idxall[s, kk, sl] = idxall[s, kk, sl] + off
            return carry

        lax.fori_loop(0, JOBW, add_off, 0)

        def wb_wait(u):
            pltpu.make_async_copy(
                rows[u], g_hbm.at[0, 0, pl.ds(0, CH_), :], wsem[u]
            ).wait()

        def block(blk, carry):
            descs = []
            for u in range(NB):
                t = blk * NB + u
                s = t // KPW_
                kk = t % KPW_

                @pl.when(blk > 0)
                def _():
                    wb_wait(u)

                descs.append(
                    pltpu.async_copy(xpk_hbm.at[idxall.at[s, kk]], rows[u], gsem[u])
                )
            for u in range(NB):
                t = blk * NB + u
                s = t // KPW_
                c = base_q + 8 * (t % KPW_)
                descs[u].wait()
                pltpu.async_copy(
                    rows[u], g_hbm.at[s, pp, pl.ds(c * CH_, CH_), :], wsem[u]
                )
            return carry

        lax.fori_loop(0, JOBW // NB, block, 0)

        for u in range(NB):
            wb_wait(u)

    return k(xpk, idxp)


def _tc_matmul(g, wt, bias, vout):
    VB = 1000

    def body(g_ref, w_ref, b_ref, o_ref):
        acc0 = jnp.zeros((VB, OUT), jnp.float32)
        acc1 = jnp.zeros((VB, OUT), jnp.float32)
        for s in range(S):
            u = lax.bitcast_convert_type(g_ref[s, 0], jnp.uint32)
            lo = lax.bitcast_convert_type(u << 16, jnp.float32)
            hi = lax.bitcast_convert_type(u & jnp.uint32(0xFFFF0000), jnp.float32)
            acc0 += jnp.dot(
                lo.astype(jnp.bfloat16), w_ref[s], preferred_element_type=jnp.float32
            )
            acc1 += jnp.dot(
                hi.astype(jnp.bfloat16), w_ref[s], preferred_element_type=jnp.float32
            )
        o_ref[0] = jnp.maximum(acc0 + b_ref[0], 0.0)
        o_ref[1] = jnp.maximum(acc1 + b_ref[0], 0.0)

    return pl.pallas_call(
        body,
        grid=(B2, vout // VB),
        in_specs=[
            pl.BlockSpec((S, 1, VB, F), lambda p, i: (0, p, i, 0)),
            pl.BlockSpec((S, F, OUT), lambda p, i: (0, 0, 0)),
            pl.BlockSpec((1, OUT), lambda p, i: (0, 0)),
        ],
        out_specs=pl.BlockSpec((2, VB, OUT), lambda p, i: (p, i, 0)),
        out_shape=jax.ShapeDtypeStruct((B, vout, OUT), jnp.float32),
        compiler_params=pltpu.CompilerParams(
            dimension_semantics=("parallel", "parallel"),
        ),
    )(g, wt, bias)


@jax.jit
def kernel(x, spiral, W, b):
    xb = lax.bitcast_convert_type(x.astype(jnp.bfloat16), jnp.uint16).astype(
        jnp.uint32
    )
    pk = xb[0::2] | (xb[1::2] << 16)
    xpk = jnp.concatenate([pk, jnp.zeros((B2, 1, F), jnp.uint32)], axis=1)
    xpk = lax.bitcast_convert_type(xpk, jnp.int32).reshape(B2 * (V + 1), F)
    idxt = spiral[0, :V, :].T
    wt = W.reshape(OUT, S, F).transpose(1, 2, 0).astype(jnp.bfloat16)
    b2 = b.reshape(1, OUT)
    gs = []
    for h in range(2):
        idxh = jnp.pad(
            idxt[:, h * VH:(h + 1) * VH], ((0, 0), (0, VPH - VH))
        ).reshape(S, KPH, 8, CH)
        gs.append(_sc_gather(xpk, idxh))
    o0 = _tc_matmul(gs[0], wt, b2, VH)
    o1 = _tc_matmul(gs[1], wt, b2, VH)
    return jnp.concatenate([o0, o1], axis=1)


# speedup vs baseline: 1.0525x; 1.0525x over previous
import functools

import jax
import jax.numpy as jnp
from jax import lax
from jax.experimental import pallas as pl
from jax.experimental.pallas import tpu as pltpu
from jax.experimental.pallas import tpu_sc as plsc

B = 8
V = 10000
S = 9
F = 128
OUT = 128

B2 = B // 2
NC = 2
NS = 16
NW = NC * NS
CH = 128
NCHUNK = 80
VP = NCHUNK * CH
NB = 6
KPW = 10
JOBW = S * KPW


def _sc_gather(xpk, idxp):
    mesh = plsc.VectorSubcoreMesh(core_axis_name="c", subcore_axis_name="s")

    @functools.partial(
        pl.kernel,
        mesh=mesh,
        out_type=jax.ShapeDtypeStruct((S, B2, VP, F), jnp.int32),
        scratch_types=[
            pltpu.VMEM((S, KPW, CH), jnp.int32),
            [pltpu.VMEM((CH, F), jnp.int32) for _ in range(NB)],
            [pltpu.SemaphoreType.DMA for _ in range(NB)],
            [pltpu.SemaphoreType.DMA for _ in range(NB)],
        ],
    )
    def k(xpk_hbm, idxp_hbm, g_hbm, idxall, rows, gsem, wsem):
        cid = lax.axis_index("c")
        sid = lax.axis_index("s")
        wid = sid * NC + cid
        pp = wid % B2
        base_q = wid // B2
        off = pp * (V + 1)

        pltpu.sync_copy(idxp_hbm.at[:, :, base_q, :], idxall)

        def add_off(a, carry):
            s = a // KPW
            kk = a % KPW
            for i in range(CH // 16):
                sl = pl.ds(i * 16, 16)
                idxall[s, kk, sl] = idxall[s, kk, sl] + off
            return carry

        lax.fori_loop(0, JOBW, add_off, 0)

        def wb_wait(u):
            pltpu.make_async_copy(
                rows[u], g_hbm.at[0, 0, pl.ds(0, CH), :], wsem[u]
            ).wait()

        def block(blk, carry):
            descs = []
            for u in range(NB):
                t = blk * NB + u
                s = t // KPW
                kk = t % KPW

                @pl.when(blk > 0)
                def _():
                    wb_wait(u)

                descs.append(
                    pltpu.async_copy(xpk_hbm.at[idxall.at[s, kk]], rows[u], gsem[u])
                )
            for u in range(NB):
                t = blk * NB + u
                s = t // KPW
                c = base_q + 8 * (t % KPW)
                descs[u].wait()
                pltpu.async_copy(
                    rows[u], g_hbm.at[s, pp, pl.ds(c * CH, CH), :], wsem[u]
                )
            return carry

        lax.fori_loop(0, JOBW // NB, block, 0)

        for u in range(NB):
            wb_wait(u)

    return k(xpk, idxp)


def _tc_matmul(g, wt, bias):
    VB = 1000

    def body(g_ref, w_ref, b_ref, o_ref):
        los, his = [], []
        for s in range(S):
            u = lax.bitcast_convert_type(g_ref[s, 0], jnp.uint32)
            lo = lax.bitcast_convert_type(u << 16, jnp.float32)
            hi = lax.bitcast_convert_type(u & jnp.uint32(0xFFFF0000), jnp.float32)
            los.append(lo.astype(jnp.bfloat16))
            his.append(hi.astype(jnp.bfloat16))
        lo = jnp.concatenate(los, axis=1)
        hi = jnp.concatenate(his, axis=1)
        acc0 = jnp.dot(lo, w_ref[...], preferred_element_type=jnp.float32)
        acc1 = jnp.dot(hi, w_ref[...], preferred_element_type=jnp.float32)
        o_ref[0] = jnp.maximum(acc0 + b_ref[0], 0.0)
        o_ref[1] = jnp.maximum(acc1 + b_ref[0], 0.0)

    return pl.pallas_call(
        body,
        grid=(B2, V // VB),
        in_specs=[
            pl.BlockSpec((S, 1, VB, F), lambda p, i: (0, p, i, 0)),
            pl.BlockSpec((S * F, OUT), lambda p, i: (0, 0)),
            pl.BlockSpec((1, OUT), lambda p, i: (0, 0)),
        ],
        out_specs=pl.BlockSpec((2, VB, OUT), lambda p, i: (p, i, 0)),
        out_shape=jax.ShapeDtypeStruct((B, V, OUT), jnp.float32),
        compiler_params=pltpu.CompilerParams(
            dimension_semantics=("parallel", "parallel"),
        ),
    )(g, wt, bias)


@jax.jit
def kernel(x, spiral, W, b):
    xb = lax.bitcast_convert_type(x.astype(jnp.bfloat16), jnp.uint16).astype(
        jnp.uint32
    )
    pk = xb[0::2] | (xb[1::2] << 16)
    xpk = jnp.concatenate([pk, jnp.zeros((B2, 1, F), jnp.uint32)], axis=1)
    xpk = lax.bitcast_convert_type(xpk, jnp.int32).reshape(B2 * (V + 1), F)
    idxp = jnp.pad(spiral[0, :V, :].T, ((0, 0), (0, VP - V))).reshape(
        S, KPW, 8, CH
    )
    g = _sc_gather(xpk, idxp)
    wt = W.T.astype(jnp.bfloat16)
    return _tc_matmul(g, wt, b.reshape(1, OUT))


# speedup vs baseline: 1.1361x; 1.0794x over previous
import functools

import jax
import jax.numpy as jnp
from jax import lax
from jax.experimental import pallas as pl
from jax.experimental.pallas import tpu as pltpu
from jax.experimental.pallas import tpu_sc as plsc

B = 8
V = 10000
S = 9
F = 128
OUT = 128

B2 = B // 2
NC = 2
NS = 16
NW = NC * NS
CH = 128
NCHUNK = 80
VP = NCHUNK * CH
NB = 6
KPW = 10
JOBW = S * KPW
PVT = 10400
PBLK = 400


def _sc_gather(xpk, idxp):
    mesh = plsc.VectorSubcoreMesh(core_axis_name="c", subcore_axis_name="s")

    @functools.partial(
        pl.kernel,
        mesh=mesh,
        out_type=jax.ShapeDtypeStruct((S, B2, VP, F), jnp.int32),
        scratch_types=[
            pltpu.VMEM((S, KPW, CH), jnp.int32),
            [pltpu.VMEM((CH, F), jnp.int32) for _ in range(NB)],
            [pltpu.SemaphoreType.DMA for _ in range(NB)],
            [pltpu.SemaphoreType.DMA for _ in range(NB)],
        ],
    )
    def k(xpk_hbm, idxp_hbm, g_hbm, idxall, rows, gsem, wsem):
        cid = lax.axis_index("c")
        sid = lax.axis_index("s")
        wid = sid * NC + cid
        pp = wid % B2
        base_q = wid // B2
        off = pp * PVT

        pltpu.sync_copy(idxp_hbm.at[:, :, base_q, :], idxall)

        def add_off(a, carry):
            s = a // KPW
            kk = a % KPW
            for i in range(CH // 16):
                sl = pl.ds(i * 16, 16)
                idxall[s, kk, sl] = idxall[s, kk, sl] + off
            return carry

        lax.fori_loop(0, JOBW, add_off, 0)

        def wb_wait(u):
            pltpu.make_async_copy(
                rows[u], g_hbm.at[0, 0, pl.ds(0, CH), :], wsem[u]
            ).wait()

        def block(blk, carry):
            descs = []
            for u in range(NB):
                t = blk * NB + u
                s = t // KPW
                kk = t % KPW

                @pl.when(blk > 0)
                def _():
                    wb_wait(u)

                descs.append(
                    pltpu.async_copy(xpk_hbm.at[idxall.at[s, kk]], rows[u], gsem[u])
                )
            for u in range(NB):
                t = blk * NB + u
                s = t // KPW
                c = base_q + 8 * (t % KPW)
                descs[u].wait()
                pltpu.async_copy(
                    rows[u], g_hbm.at[s, pp, pl.ds(c * CH, CH), :], wsem[u]
                )
            return carry

        lax.fori_loop(0, JOBW // NB, block, 0)

        for u in range(NB):
            wb_wait(u)

    return k(xpk, idxp)


def _tc_pack(x):
    nxb = V // PBLK

    def body(x_ref, o_ref):
        i = pl.program_id(0)
        rows = i * PBLK + lax.broadcasted_iota(jnp.int32, (PBLK, F), 0)
        valid = rows < V
        for p in range(B2):
            a = lax.bitcast_convert_type(
                x_ref[2 * p].astype(jnp.bfloat16), jnp.uint16
            ).astype(jnp.uint32)
            c = lax.bitcast_convert_type(
                x_ref[2 * p + 1].astype(jnp.bfloat16), jnp.uint16
            ).astype(jnp.uint32)
            w = jnp.where(valid, a | (c << 16), jnp.uint32(0))
            o_ref[p] = lax.bitcast_convert_type(w, jnp.int32)

    return pl.pallas_call(
        body,
        grid=(PVT // PBLK,),
        in_specs=[
            pl.BlockSpec((B, PBLK, F), lambda i: (0, jnp.minimum(i, nxb - 1), 0))
        ],
        out_specs=pl.BlockSpec((B2, PBLK, F), lambda i: (0, i, 0)),
        out_shape=jax.ShapeDtypeStruct((B2, PVT, F), jnp.int32),
        compiler_params=pltpu.CompilerParams(
            dimension_semantics=("arbitrary",),
        ),
    )(x)


def _tc_matmul(g, wt, bias):
    VB = 1000

    def body(g_ref, w_ref, b_ref, o_ref):
        los, his = [], []
        for s in range(S):
            u = lax.bitcast_convert_type(g_ref[s, 0], jnp.uint32)
            lo = lax.bitcast_convert_type(u << 16, jnp.float32)
            hi = lax.bitcast_convert_type(u & jnp.uint32(0xFFFF0000), jnp.float32)
            los.append(lo.astype(jnp.bfloat16))
            his.append(hi.astype(jnp.bfloat16))
        lo = jnp.concatenate(los, axis=1)
        hi = jnp.concatenate(his, axis=1)
        acc0 = jnp.dot(lo, w_ref[...], preferred_element_type=jnp.float32)
        acc1 = jnp.dot(hi, w_ref[...], preferred_element_type=jnp.float32)
        o_ref[0] = jnp.maximum(acc0 + b_ref[0], 0.0)
        o_ref[1] = jnp.maximum(acc1 + b_ref[0], 0.0)

    return pl.pallas_call(
        body,
        grid=(B2, V // VB),
        in_specs=[
            pl.BlockSpec((S, 1, VB, F), lambda p, i: (0, p, i, 0)),
            pl.BlockSpec((S * F, OUT), lambda p, i: (0, 0)),
            pl.BlockSpec((1, OUT), lambda p, i: (0, 0)),
        ],
        out_specs=pl.BlockSpec((2, VB, OUT), lambda p, i: (p, i, 0)),
        out_shape=jax.ShapeDtypeStruct((B, V, OUT), jnp.float32),
        compiler_params=pltpu.CompilerParams(
            dimension_semantics=("parallel", "parallel"),
        ),
    )(g, wt, bias)


@jax.jit
def kernel(x, spiral, W, b):
    xpk = _tc_pack(x).reshape(B2 * PVT, F)
    idxp = jnp.pad(spiral[0, :V, :].T, ((0, 0), (0, VP - V))).reshape(
        S, KPW, 8, CH
    )
    g = _sc_gather(xpk, idxp)
    wt = W.T.astype(jnp.bfloat16)
    return _tc_matmul(g, wt, b.reshape(1, OUT))


# speedup vs baseline: 1.9703x; 1.7342x over previous
import functools

import jax
import jax.numpy as jnp
from jax import lax
from jax.experimental import pallas as pl
from jax.experimental.pallas import tpu as pltpu
from jax.experimental.pallas import tpu_sc as plsc

B = 8
V = 10000
S = 9
F = 128
OUT = 128

B2 = B // 2
NC = 2
NS = 16
NW = NC * NS
CH = 128
NCHUNK = 80
VP = NCHUNK * CH
NB = 6
KPW = 10
JOBW = S * KPW
PVT = 10240
PBLK = 512


CH2 = 64
KP2 = 5
NB2 = 3
JOB2 = S * KP2
TLOAD = PVT // NS


def _sc_gather(xpk, idxp):
    mesh = plsc.VectorSubcoreMesh(core_axis_name="c", subcore_axis_name="s")

    @functools.partial(
        pl.kernel,
        mesh=mesh,
        out_type=jax.ShapeDtypeStruct((S, B2, VP, F), jnp.int32),
        scratch_types=[
            pltpu.VMEM_SHARED((PVT, F), jnp.int32),
            pltpu.VMEM((S, KP2, CH2), jnp.int32),
            [pltpu.VMEM((CH2, F), jnp.int32) for _ in range(NB2)],
            pltpu.SemaphoreType.DMA,
            [pltpu.SemaphoreType.DMA for _ in range(NB2)],
            [pltpu.SemaphoreType.DMA for _ in range(NB2)],
        ],
    )
    def k(xpk_hbm, idxp_hbm, g_hbm, shared, idxall, rows, lsem, gsem, wsem):
        cid = lax.axis_index("c")
        sid = lax.axis_index("s")
        wid = sid * NC + cid

        pltpu.sync_copy(idxp_hbm.at[:, :, wid, :], idxall)

        def wb_wait(u):
            pltpu.make_async_copy(
                rows[u], g_hbm.at[0, 0, pl.ds(0, CH2), :], wsem[u]
            ).wait()

        for p in range(B2):
            pltpu.async_copy(
                xpk_hbm.at[pl.ds(p * PVT + sid * TLOAD, TLOAD), :],
                shared.at[pl.ds(sid * TLOAD, TLOAD), :],
                lsem,
            ).wait()
            plsc.subcore_barrier()

            def block(blk, carry):
                descs = []
                for u in range(NB2):
                    t = blk * NB2 + u
                    s = t // KP2
                    kk = t % KP2

                    if p == 0:
                        @pl.when(blk > 0)
                        def _():
                            wb_wait(u)
                    else:
                        wb_wait(u)

                    descs.append(
                        pltpu.async_copy(
                            shared.at[idxall.at[s, kk]], rows[u], gsem[u]
                        )
                    )
                for u in range(NB2):
                    t = blk * NB2 + u
                    s = t // KP2
                    c = wid + NW * (t % KP2)
                    descs[u].wait()
                    pltpu.async_copy(
                        rows[u], g_hbm.at[s, p, pl.ds(c * CH2, CH2), :], wsem[u]
                    )
                return carry

            lax.fori_loop(0, JOB2 // NB2, block, 0)
            plsc.subcore_barrier()

        for u in range(NB2):
            wb_wait(u)

    return k(xpk, idxp)


def _tc_pack(x):
    def body(x_ref, o_ref):
        i = pl.program_id(0)
        rows = i * PBLK + lax.broadcasted_iota(jnp.int32, (PBLK, F), 0)
        valid = rows < V
        for p in range(B2):
            a = lax.bitcast_convert_type(
                x_ref[2 * p].astype(jnp.bfloat16), jnp.uint16
            ).astype(jnp.uint32)
            c = lax.bitcast_convert_type(
                x_ref[2 * p + 1].astype(jnp.bfloat16), jnp.uint16
            ).astype(jnp.uint32)
            w = jnp.where(valid, a | (c << 16), jnp.uint32(0))
            o_ref[p] = lax.bitcast_convert_type(w, jnp.int32)

    return pl.pallas_call(
        body,
        grid=(PVT // PBLK,),
        in_specs=[
            pl.BlockSpec((B, PBLK, F), lambda i: (0, i, 0))
        ],
        out_specs=pl.BlockSpec((B2, PBLK, F), lambda i: (0, i, 0)),
        out_shape=jax.ShapeDtypeStruct((B2, PVT, F), jnp.int32),
        compiler_params=pltpu.CompilerParams(
            dimension_semantics=("arbitrary",),
        ),
    )(x)


def _tc_matmul(g, wt, bias):
    VB = 1000

    def body(g_ref, w_ref, b_ref, o_ref):
        los, his = [], []
        for s in range(S):
            u = lax.bitcast_convert_type(g_ref[s, 0], jnp.uint32)
            lo = lax.bitcast_convert_type(u << 16, jnp.float32)
            hi = lax.bitcast_convert_type(u & jnp.uint32(0xFFFF0000), jnp.float32)
            los.append(lo.astype(jnp.bfloat16))
            his.append(hi.astype(jnp.bfloat16))
        lo = jnp.concatenate(los, axis=1)
        hi = jnp.concatenate(his, axis=1)
        acc0 = jnp.dot(lo, w_ref[...], preferred_element_type=jnp.float32)
        acc1 = jnp.dot(hi, w_ref[...], preferred_element_type=jnp.float32)
        o_ref[0] = jnp.maximum(acc0 + b_ref[0], 0.0)
        o_ref[1] = jnp.maximum(acc1 + b_ref[0], 0.0)

    return pl.pallas_call(
        body,
        grid=(B2, V // VB),
        in_specs=[
            pl.BlockSpec((S, 1, VB, F), lambda p, i: (0, p, i, 0)),
            pl.BlockSpec((S * F, OUT), lambda p, i: (0, 0)),
            pl.BlockSpec((1, OUT), lambda p, i: (0, 0)),
        ],
        out_specs=pl.BlockSpec((2, VB, OUT), lambda p, i: (p, i, 0)),
        out_shape=jax.ShapeDtypeStruct((B, V, OUT), jnp.float32),
        compiler_params=pltpu.CompilerParams(
            dimension_semantics=("parallel", "parallel"),
        ),
    )(g, wt, bias)


@jax.jit
def kernel(x, spiral, W, b):
    xpk = _tc_pack(x).reshape(B2 * PVT, F)
    idxp = jnp.pad(spiral[0, :V, :].T, ((0, 0), (0, VP - V))).reshape(
        S, KP2, NW, CH2
    )
    g = _sc_gather(xpk, idxp)
    wt = W.T.astype(jnp.bfloat16)
    return _tc_matmul(g, wt, b.reshape(1, OUT))


# speedup vs baseline: 2.0728x; 1.0521x over previous
import functools

import jax
import jax.numpy as jnp
from jax import lax
from jax.experimental import pallas as pl
from jax.experimental.pallas import tpu as pltpu
from jax.experimental.pallas import tpu_sc as plsc

B = 8
V = 10000
S = 9
F = 128
OUT = 128

B2 = B // 2
NC = 2
NS = 16
NW = NC * NS
CH = 128
NCHUNK = 80
VP = NCHUNK * CH
NB = 6
KPW = 10
JOBW = S * KPW
PVT = 10240
PBLK = 512


CH2 = 64
KP2 = 5
NB2 = 3
JOB2 = S * KP2
TLOAD = PVT // NS


def _sc_gather(xpk, idxp):
    mesh = plsc.VectorSubcoreMesh(core_axis_name="c", subcore_axis_name="s")

    @functools.partial(
        pl.kernel,
        mesh=mesh,
        out_type=jax.ShapeDtypeStruct((S, B2, VP, F), jnp.int32),
        scratch_types=[
            pltpu.VMEM_SHARED((PVT, F), jnp.int32),
            pltpu.VMEM((S, KP2, CH2), jnp.int32),
            [pltpu.VMEM((CH2, F), jnp.int32) for _ in range(NB2)],
            pltpu.SemaphoreType.DMA,
            [pltpu.SemaphoreType.DMA for _ in range(NB2)],
            [pltpu.SemaphoreType.DMA for _ in range(NB2)],
        ],
    )
    def k(xpk_hbm, idxp_hbm, g_hbm, shared, idxall, rows, lsem, gsem, wsem):
        cid = lax.axis_index("c")
        sid = lax.axis_index("s")
        wid = sid * NC + cid

        pltpu.sync_copy(idxp_hbm.at[:, :, wid, :], idxall)

        def wb_wait(u):
            pltpu.make_async_copy(
                rows[u], g_hbm.at[0, 0, pl.ds(0, CH2), :], wsem[u]
            ).wait()

        for p in range(B2):
            pltpu.async_copy(
                xpk_hbm.at[pl.ds(p * PVT + sid * TLOAD, TLOAD), :],
                shared.at[pl.ds(sid * TLOAD, TLOAD), :],
                lsem,
            ).wait()
            plsc.subcore_barrier()

            def block(blk, carry):
                descs = []
                for u in range(NB2):
                    t = blk * NB2 + u
                    s = t // KP2
                    kk = t % KP2

                    if p == 0:
                        @pl.when(blk > 0)
                        def _():
                            wb_wait(u)
                    else:
                        wb_wait(u)

                    descs.append(
                        pltpu.async_copy(
                            shared.at[idxall.at[s, kk]], rows[u], gsem[u]
                        )
                    )
                for u in range(NB2):
                    t = blk * NB2 + u
                    s = t // KP2
                    c = wid + NW * (t % KP2)
                    descs[u].wait()
                    pltpu.async_copy(
                        rows[u], g_hbm.at[s, p, pl.ds(c * CH2, CH2), :], wsem[u]
                    )
                return carry

            lax.fori_loop(0, JOB2 // NB2, block, 0)
            plsc.subcore_barrier()

        for u in range(NB2):
            wb_wait(u)

    return k(xpk, idxp)


def _tc_pack(x):
    def body(x_ref, o_ref):
        i = pl.program_id(0)
        rows = i * PBLK + lax.broadcasted_iota(jnp.int32, (PBLK, F), 0)
        valid = rows < V
        for p in range(B2):
            a = lax.bitcast_convert_type(
                x_ref[2 * p].astype(jnp.bfloat16), jnp.uint16
            ).astype(jnp.uint32)
            c = lax.bitcast_convert_type(
                x_ref[2 * p + 1].astype(jnp.bfloat16), jnp.uint16
            ).astype(jnp.uint32)
            w = jnp.where(valid, a | (c << 16), jnp.uint32(0))
            o_ref[p] = lax.bitcast_convert_type(w, jnp.int32)

    return pl.pallas_call(
        body,
        grid=(PVT // PBLK,),
        in_specs=[
            pl.BlockSpec((B, PBLK, F), lambda i: (0, i, 0))
        ],
        out_specs=pl.BlockSpec((B2, PBLK, F), lambda i: (0, i, 0)),
        out_shape=jax.ShapeDtypeStruct((B2, PVT, F), jnp.int32),
        compiler_params=pltpu.CompilerParams(
            dimension_semantics=("arbitrary",),
        ),
    )(x)


def _tc_matmul(g, wt, bias):
    VB = 2000

    def body(g_ref, w_ref, b_ref, o_ref):
        los, his = [], []
        for s in range(S):
            u = lax.bitcast_convert_type(g_ref[s, 0], jnp.uint32)
            lo = lax.bitcast_convert_type(u << 16, jnp.float32)
            hi = lax.bitcast_convert_type(u & jnp.uint32(0xFFFF0000), jnp.float32)
            los.append(lo.astype(jnp.bfloat16))
            his.append(hi.astype(jnp.bfloat16))
        lo = jnp.concatenate(los, axis=1)
        hi = jnp.concatenate(his, axis=1)
        acc0 = jnp.dot(lo, w_ref[...], preferred_element_type=jnp.float32)
        acc1 = jnp.dot(hi, w_ref[...], preferred_element_type=jnp.float32)
        o_ref[0] = jnp.maximum(acc0 + b_ref[0], 0.0)
        o_ref[1] = jnp.maximum(acc1 + b_ref[0], 0.0)

    return pl.pallas_call(
        body,
        grid=(B2, V // VB),
        in_specs=[
            pl.BlockSpec((S, 1, VB, F), lambda p, i: (0, p, i, 0)),
            pl.BlockSpec((S * F, OUT), lambda p, i: (0, 0)),
            pl.BlockSpec((1, OUT), lambda p, i: (0, 0)),
        ],
        out_specs=pl.BlockSpec((2, VB, OUT), lambda p, i: (p, i, 0)),
        out_shape=jax.ShapeDtypeStruct((B, V, OUT), jnp.float32),
        compiler_params=pltpu.CompilerParams(
            dimension_semantics=("parallel", "parallel"),
        ),
    )(g, wt, bias)


@jax.jit
def kernel(x, spiral, W, b):
    xpk = _tc_pack(x).reshape(B2 * PVT, F)
    idxp = jnp.pad(spiral[0, :V, :].T, ((0, 0), (0, VP - V))).reshape(
        S, KP2, NW, CH2
    )
    g = _sc_gather(xpk, idxp)
    wt = W.T.astype(jnp.bfloat16)
    return _tc_matmul(g, wt, b.reshape(1, OUT))


# speedup vs baseline: 2.1370x; 1.0310x over previous
import functools

import jax
import jax.numpy as jnp
from jax import lax
from jax.experimental import pallas as pl
from jax.experimental.pallas import tpu as pltpu
from jax.experimental.pallas import tpu_sc as plsc

B = 8
V = 10000
S = 9
F = 128
OUT = 128

B2 = B // 2
NC = 2
NS = 16
NW = NC * NS
CH = 128
NCHUNK = 80
VP = NCHUNK * CH
NB = 6
KPW = 10
JOBW = S * KPW
PVT = 10112
PBLK = 632


CH2 = 64
KP2 = 5
NB2 = 5
JOB2 = S * KP2
TLOAD = PVT // NS


def _sc_gather(xpk, idxp):
    mesh = plsc.VectorSubcoreMesh(core_axis_name="c", subcore_axis_name="s")

    @functools.partial(
        pl.kernel,
        mesh=mesh,
        out_type=jax.ShapeDtypeStruct((S, B2, VP, F), jnp.int32),
        scratch_types=[
            pltpu.VMEM_SHARED((PVT, F), jnp.int32),
            pltpu.VMEM((S, KP2, CH2), jnp.int32),
            [pltpu.VMEM((CH2, F), jnp.int32) for _ in range(NB2)],
            pltpu.SemaphoreType.DMA,
            [pltpu.SemaphoreType.DMA for _ in range(NB2)],
            [pltpu.SemaphoreType.DMA for _ in range(NB2)],
        ],
    )
    def k(xpk_hbm, idxp_hbm, g_hbm, shared, idxall, rows, lsem, gsem, wsem):
        cid = lax.axis_index("c")
        sid = lax.axis_index("s")
        wid = sid * NC + cid

        pltpu.sync_copy(idxp_hbm.at[:, :, wid, :], idxall)

        def wb_wait(u):
            pltpu.make_async_copy(
                rows[u], g_hbm.at[0, 0, pl.ds(0, CH2), :], wsem[u]
            ).wait()

        for p in range(B2):
            pltpu.async_copy(
                xpk_hbm.at[pl.ds(p * PVT + sid * TLOAD, TLOAD), :],
                shared.at[pl.ds(sid * TLOAD, TLOAD), :],
                lsem,
            ).wait()
            plsc.subcore_barrier()

            def block(blk, carry):
                descs = []
                for u in range(NB2):
                    t = blk * NB2 + u
                    s = t // KP2
                    kk = t % KP2

                    if p == 0:
                        @pl.when(blk > 0)
                        def _():
                            wb_wait(u)
                    else:
                        wb_wait(u)

                    descs.append(
                        pltpu.async_copy(
                            shared.at[idxall.at[s, kk]], rows[u], gsem[u]
                        )
                    )
                for u in range(NB2):
                    t = blk * NB2 + u
                    s = t // KP2
                    c = wid + NW * (t % KP2)
                    descs[u].wait()
                    pltpu.async_copy(
                        rows[u], g_hbm.at[s, p, pl.ds(c * CH2, CH2), :], wsem[u]
                    )
                return carry

            lax.fori_loop(0, JOB2 // NB2, block, 0)
            plsc.subcore_barrier()

        for u in range(NB2):
            wb_wait(u)

    return k(xpk, idxp)


def _tc_pack(x):
    def body(x_ref, o_ref):
        i = pl.program_id(0)
        rows = i * PBLK + lax.broadcasted_iota(jnp.int32, (PBLK, F), 0)
        valid = rows < V
        for p in range(B2):
            a = lax.bitcast_convert_type(
                x_ref[2 * p].astype(jnp.bfloat16), jnp.uint16
            ).astype(jnp.uint32)
            c = lax.bitcast_convert_type(
                x_ref[2 * p + 1].astype(jnp.bfloat16), jnp.uint16
            ).astype(jnp.uint32)
            w = jnp.where(valid, a | (c << 16), jnp.uint32(0))
            o_ref[p] = lax.bitcast_convert_type(w, jnp.int32)

    return pl.pallas_call(
        body,
        grid=(PVT // PBLK,),
        in_specs=[
            pl.BlockSpec((B, PBLK, F), lambda i: (0, i, 0))
        ],
        out_specs=pl.BlockSpec((B2, PBLK, F), lambda i: (0, i, 0)),
        out_shape=jax.ShapeDtypeStruct((B2, PVT, F), jnp.int32),
        compiler_params=pltpu.CompilerParams(
            dimension_semantics=("arbitrary",),
        ),
    )(x)


def _tc_matmul(g, wt, bias):
    VB = 2000

    def body(g_ref, w_ref, b_ref, o_ref):
        los, his = [], []
        for s in range(S):
            u = lax.bitcast_convert_type(g_ref[s, 0], jnp.uint32)
            lo = lax.bitcast_convert_type(u << 16, jnp.float32)
            hi = lax.bitcast_convert_type(u & jnp.uint32(0xFFFF0000), jnp.float32)
            los.append(lo.astype(jnp.bfloat16))
            his.append(hi.astype(jnp.bfloat16))
        lo = jnp.concatenate(los, axis=1)
        hi = jnp.concatenate(his, axis=1)
        acc0 = jnp.dot(lo, w_ref[...], preferred_element_type=jnp.float32)
        acc1 = jnp.dot(hi, w_ref[...], preferred_element_type=jnp.float32)
        o_ref[0] = jnp.maximum(acc0 + b_ref[0], 0.0)
        o_ref[1] = jnp.maximum(acc1 + b_ref[0], 0.0)

    return pl.pallas_call(
        body,
        grid=(B2, V // VB),
        in_specs=[
            pl.BlockSpec((S, 1, VB, F), lambda p, i: (0, p, i, 0)),
            pl.BlockSpec((S * F, OUT), lambda p, i: (0, 0)),
            pl.BlockSpec((1, OUT), lambda p, i: (0, 0)),
        ],
        out_specs=pl.BlockSpec((2, VB, OUT), lambda p, i: (p, i, 0)),
        out_shape=jax.ShapeDtypeStruct((B, V, OUT), jnp.float32),
        compiler_params=pltpu.CompilerParams(
            dimension_semantics=("parallel", "parallel"),
        ),
    )(g, wt, bias)


@jax.jit
def kernel(x, spiral, W, b):
    xpk = _tc_pack(x).reshape(B2 * PVT, F)
    idxp = jnp.pad(spiral[0, :V, :].T, ((0, 0), (0, VP - V))).reshape(
        S, KP2, NW, CH2
    )
    g = _sc_gather(xpk, idxp)
    wt = W.T.astype(jnp.bfloat16)
    return _tc_matmul(g, wt, b.reshape(1, OUT))


# speedup vs baseline: 2.1405x; 1.0016x over previous
import functools

import jax
import jax.numpy as jnp
from jax import lax
from jax.experimental import pallas as pl
from jax.experimental.pallas import tpu as pltpu
from jax.experimental.pallas import tpu_sc as plsc

B = 8
V = 10000
S = 9
F = 128
OUT = 128

B2 = B // 2
NC = 2
NS = 16
NW = NC * NS
VP = 10240
PVT = 10112
PBLK = 632
CH2 = 64
KP2 = 5
NB2 = 5
JOB2 = S * KP2
TLOAD = PVT // NS


def _sc_gather(xpk, idxp):
    mesh = plsc.VectorSubcoreMesh(core_axis_name="c", subcore_axis_name="s")

    @functools.partial(
        pl.kernel,
        mesh=mesh,
        out_type=jax.ShapeDtypeStruct((S, B2, VP, F), jnp.int32),
        scratch_types=[
            pltpu.VMEM_SHARED((PVT, F), jnp.int32),
            pltpu.VMEM((S, KP2, CH2), jnp.int32),
            [pltpu.VMEM((CH2, F), jnp.int32) for _ in range(NB2)],
            pltpu.SemaphoreType.DMA,
            [pltpu.SemaphoreType.DMA for _ in range(NB2)],
            [pltpu.SemaphoreType.DMA for _ in range(NB2)],
        ],
    )
    def k(xpk_hbm, idxp_hbm, g_hbm, shared, idxall, rows, lsem, gsem, wsem):
        cid = lax.axis_index("c")
        sid = lax.axis_index("s")
        wid = sid * NC + cid

        pltpu.sync_copy(idxp_hbm.at[:, :, wid, :], idxall)

        def wb_wait(u):
            pltpu.make_async_copy(
                rows[u], g_hbm.at[0, 0, pl.ds(0, CH2), :], wsem[u]
            ).wait()

        for p in range(B2):
            pltpu.async_copy(
                xpk_hbm.at[pl.ds(p * PVT + sid * TLOAD, TLOAD), :],
                shared.at[pl.ds(sid * TLOAD, TLOAD), :],
                lsem,
            ).wait()
            plsc.subcore_barrier()

            def block(blk, carry):
                descs = []
                for u in range(NB2):
                    t = blk * NB2 + u
                    s = t // KP2
                    kk = t % KP2

                    if p == 0:
                        @pl.when(blk > 0)
                        def _():
                            wb_wait(u)
                    else:
                        wb_wait(u)

                    descs.append(
                        pltpu.async_copy(
                            shared.at[idxall.at[s, kk]], rows[u], gsem[u]
                        )
                    )
                for u in range(NB2):
                    t = blk * NB2 + u
                    s = t // KP2
                    c = wid + NW * (t % KP2)
                    descs[u].wait()
                    pltpu.async_copy(
                        rows[u], g_hbm.at[s, p, pl.ds(c * CH2, CH2), :], wsem[u]
                    )
                return carry

            lax.fori_loop(0, JOB2 // NB2, block, 0)
            plsc.subcore_barrier()

        for u in range(NB2):
            wb_wait(u)

    return k(xpk, idxp)


def _tc_pack(x):
    def body(x_ref, o_ref):
        i = pl.program_id(0)
        rows = i * PBLK + lax.broadcasted_iota(jnp.int32, (PBLK, F), 0)
        valid = rows < V
        for p in range(B2):
            a = lax.bitcast_convert_type(
                x_ref[2 * p].astype(jnp.bfloat16), jnp.uint16
            ).astype(jnp.uint32)
            c = lax.bitcast_convert_type(
                x_ref[2 * p + 1].astype(jnp.bfloat16), jnp.uint16
            ).astype(jnp.uint32)
            w = jnp.where(valid, a | (c << 16), jnp.uint32(0))
            o_ref[p] = lax.bitcast_convert_type(w, jnp.int32)

    return pl.pallas_call(
        body,
        grid=(PVT // PBLK,),
        in_specs=[
            pl.BlockSpec((B, PBLK, F), lambda i: (0, i, 0))
        ],
        out_specs=pl.BlockSpec((B2, PBLK, F), lambda i: (0, i, 0)),
        out_shape=jax.ShapeDtypeStruct((B2, PVT, F), jnp.int32),
        compiler_params=pltpu.CompilerParams(
            dimension_semantics=("arbitrary",),
        ),
    )(x)


def _tc_matmul(g, wt, bias):
    VB = 2000

    def body(g_ref, w_ref, b_ref, o_ref):
        los, his = [], []
        for s in range(S):
            u = lax.bitcast_convert_type(g_ref[s, 0], jnp.uint32)
            lo = lax.bitcast_convert_type(u << 16, jnp.float32)
            hi = lax.bitcast_convert_type(u & jnp.uint32(0xFFFF0000), jnp.float32)
            los.append(lo.astype(jnp.bfloat16))
            his.append(hi.astype(jnp.bfloat16))
        lo = jnp.concatenate(los, axis=1)
        hi = jnp.concatenate(his, axis=1)
        acc0 = jnp.dot(lo, w_ref[...], preferred_element_type=jnp.float32)
        acc1 = jnp.dot(hi, w_ref[...], preferred_element_type=jnp.float32)
        o_ref[0] = jnp.maximum(acc0 + b_ref[0], 0.0)
        o_ref[1] = jnp.maximum(acc1 + b_ref[0], 0.0)

    return pl.pallas_call(
        body,
        grid=(B2, V // VB),
        in_specs=[
            pl.BlockSpec((S, 1, VB, F), lambda p, i: (0, p, i, 0)),
            pl.BlockSpec((S * F, OUT), lambda p, i: (0, 0)),
            pl.BlockSpec((1, OUT), lambda p, i: (0, 0)),
        ],
        out_specs=pl.BlockSpec((2, VB, OUT), lambda p, i: (p, i, 0)),
        out_shape=jax.ShapeDtypeStruct((B, V, OUT), jnp.float32),
        compiler_params=pltpu.CompilerParams(
            dimension_semantics=("parallel", "parallel"),
        ),
    )(g, wt, bias)


@jax.jit
def kernel(x, spiral, W, b):
    xpk = _tc_pack(x).reshape(B2 * PVT, F)
    idxp = jnp.pad(spiral[0, :V, :].T, ((0, 0), (0, VP - V))).reshape(
        S, KP2, NW, CH2
    )
    g = _sc_gather(xpk, idxp)
    wt = W.T.astype(jnp.bfloat16)
    return _tc_matmul(g, wt, b.reshape(1, OUT))
